# baseline (speedup 1.0000x reference)
import functools

import jax
import jax.numpy as jnp
from jax import lax
from jax.experimental import pallas as pl
from jax.experimental.pallas import tpu as pltpu
from jax.experimental.pallas import tpu_sc as plsc

_OUT_H = 64
_OUT_W = 64
_NUM_NEURONS = _OUT_H * _OUT_W

_WIN_ROWS = 8
_WIN = _WIN_ROWS * _OUT_W
_NEGLIGIBLE_Z = 40.0


_BLK = 64


def _dot_hi(a, b):
    return jax.lax.dot_general(a, b, (((1,), (0,)), ((), ())),
                               precision=jax.lax.Precision.HIGHEST,
                               preferred_element_type=jnp.float32)


def _dsom_step_body(xT_ref, xblk_ref, nT_ref, lr_ref, es_ref,
                    locs_ref, bmu_ref, nout_ref, n_scr, norm_scr, p_scr):
    t = pl.program_id(0)

    @pl.when(t == 0)
    def _init():
        nt = nT_ref[...]
        n_scr[...] = nt
        norm_scr[...] = jnp.sum(nt * nt, axis=0, keepdims=True)

    xblk = xblk_ref[...]

    @pl.when(t % _BLK == 0)
    def _refresh_panel():
        p_scr[...] = _dot_hi(xblk, n_scr[...])

    tm = t % _BLK
    x = xT_ref[0]
    xrow = xblk_ref[pl.ds(tm, 1), :]
    n = n_scr[...]

    nx = p_scr[pl.ds(tm, 1), :]
    xnorm = jnp.sum(xrow * xrow)
    d2 = jnp.maximum(norm_scr[...] - 2.0 * nx + xnorm, 0.0)

    m = jnp.min(d2)
    lane = lax.broadcasted_iota(jnp.int32, (1, _NUM_NEURONS), 1)
    bmu = jnp.min(jnp.where(d2 == m, lane, _NUM_NEURONS))
    row = bmu >> 6
    col = bmu & (_OUT_W - 1)

    lr = lr_ref[0, 0]
    es_m = es_ref[0, 0] * m
    r_max = jnp.sqrt(_NEGLIGIBLE_Z * es_m).astype(jnp.int32)
    fits = jnp.logical_and(m > jnp.float32(0.0),
                           2 * r_max + 2 <= _WIN_ROWS)

    def _neigh_update(sub_lane, nw, d2w):
        kr = sub_lane >> 6
        kc = sub_lane & (_OUT_W - 1)
        gd = (jnp.abs(row - kr) + jnp.abs(col - kc)).astype(jnp.float32)
        neigh = jnp.where(m == jnp.float32(0.0),
                          jnp.zeros_like(d2w),
                          jnp.exp(-(gd * gd) / es_m))
        c = lr * (jnp.sqrt(d2w) * neigh)
        return nw + c * (x - nw)

    @pl.when(fits)
    def _windowed_update():
        r0 = jnp.clip(row - r_max, 0, _OUT_H - _WIN_ROWS) & ~1
        s = pl.multiple_of(r0 * _OUT_W, 2 * _OUT_W)
        lanes = lax.broadcasted_iota(jnp.int32, (1, _WIN), 1) + s
        nw = n_scr[:, pl.ds(s, _WIN)]
        nxw = p_scr[pl.ds(tm, 1), pl.ds(s, _WIN)]
        d2w = jnp.maximum(norm_scr[:, pl.ds(s, _WIN)] - 2.0 * nxw + xnorm,
                          0.0)
        nw_new = _neigh_update(lanes, nw, d2w)
        n_scr[:, pl.ds(s, _WIN)] = nw_new
        norm_scr[:, pl.ds(s, _WIN)] = jnp.sum(nw_new * nw_new, axis=0,
                                              keepdims=True)
        corr = _dot_hi(xblk, nw_new - nw)
        p_scr[:, pl.ds(s, _WIN)] = p_scr[:, pl.ds(s, _WIN)] + corr

    @pl.when(jnp.logical_not(fits))
    def _dense_update():
        n_new = _neigh_update(lane, n, d2)
        n_scr[...] = n_new
        norm_scr[...] = jnp.sum(n_new * n_new, axis=0, keepdims=True)
        p_scr[...] = _dot_hi(xblk, n_new)

    locs_ref[0, 0, 0] = row
    locs_ref[0, 0, 1] = col
    bmu_ref[0, 0, 0] = bmu

    @pl.when(t == pl.num_programs(0) - 1)
    def _finish():
        nout_ref[...] = n_scr[...].T


def _dsom_scan(x3, xrows, nT, lr, es):
    b, d, _ = x3.shape
    k = nT.shape[1]
    return pl.pallas_call(
        _dsom_step_body,
        grid=(b,),
        in_specs=[
            pl.BlockSpec((1, d, 1), lambda t: (t, 0, 0)),
            pl.BlockSpec((_BLK, d), lambda t: (t // _BLK, 0)),
            pl.BlockSpec((d, k), lambda t: (0, 0)),
            pl.BlockSpec(memory_space=pltpu.SMEM),
            pl.BlockSpec(memory_space=pltpu.SMEM),
        ],
        out_specs=[
            pl.BlockSpec((1, 1, 2), lambda t: (t, 0, 0), memory_space=pltpu.SMEM),
            pl.BlockSpec((1, 1, 1), lambda t: (t, 0, 0), memory_space=pltpu.SMEM),
            pl.BlockSpec((k, d), lambda t: (0, 0)),
        ],
        out_shape=[
            jax.ShapeDtypeStruct((b, 1, 2), jnp.int32),
            jax.ShapeDtypeStruct((b, 1, 1), jnp.int32),
            jax.ShapeDtypeStruct((k, d), jnp.float32),
        ],
        scratch_shapes=[pltpu.VMEM((d, k), jnp.float32),
                        pltpu.VMEM((1, k), jnp.float32),
                        pltpu.VMEM((_BLK, k), jnp.float32)],
    )(x3, xrows, nT, lr, es)


def _sc_gather(table, idx):
    info = plsc.get_sparse_core_info()
    nw = info.num_cores * info.num_subcores
    b = idx.shape[0]
    d = table.shape[1]
    b_per_w = b // nw
    mesh = plsc.VectorSubcoreMesh(core_axis_name="c", subcore_axis_name="s")

    @functools.partial(
        pl.kernel, mesh=mesh,
        out_type=jax.ShapeDtypeStruct((b, d), jnp.float32),
        scratch_types=[
            pltpu.VMEM((b_per_w,), jnp.int32),
            pltpu.VMEM((b_per_w, d), jnp.float32),
            pltpu.SemaphoreType.DMA,
        ],
    )
    def gather_kernel(table_hbm, idx_hbm, out_hbm, idx_v, rows_v, sem):
        wid = lax.axis_index("s") * info.num_cores + lax.axis_index("c")
        base = wid * b_per_w
        pltpu.sync_copy(idx_hbm.at[pl.ds(base, b_per_w)], idx_v)
        pltpu.async_copy(table_hbm.at[idx_v], rows_v, sem).wait()
        pltpu.sync_copy(rows_v, out_hbm.at[pl.ds(base, b_per_w)])

    return gather_kernel(table, idx)


@jax.jit
def kernel(input, neurons, learning_rate, elasticity_squared):
    b, d = input.shape
    x3 = input.reshape(b, d, 1)
    nT = neurons.T
    lr = jnp.asarray(learning_rate, jnp.float32).reshape(1, 1)
    es = jnp.asarray(elasticity_squared, jnp.float32).reshape(1, 1)

    locs, bmus, n_final = _dsom_scan(x3, input, nT, lr, es)
    values = _sc_gather(n_final, bmus.reshape(b))
    return locs.reshape(b, 2), values


# speedup vs baseline: 1.2379x; 1.2379x over previous
import functools

import jax
import jax.numpy as jnp
from jax import lax
from jax.experimental import pallas as pl
from jax.experimental.pallas import tpu as pltpu
from jax.experimental.pallas import tpu_sc as plsc

_OUT_H = 64
_OUT_W = 64
_NUM_NEURONS = _OUT_H * _OUT_W

_WIN_ROWS = 4
_WIN = _WIN_ROWS * _OUT_W
_NEGLIGIBLE_Z = 40.0


def _dsom_step_body(xp_ref, xn_ref, nT_ref, lr_ref, es_ref,
                    locs_ref, bmu_ref, nout_ref,
                    n_scr, norm_scr, nx_scr, d2_scr):
    i = pl.program_id(0)

    @pl.when(i == 0)
    def _init():
        nt = nT_ref[...]
        n_scr[...] = nt
        norm_scr[...] = jnp.sum(nt * nt, axis=0, keepdims=True)
        nx_scr[...] = jnp.sum(xp_ref[0][:, 0:1] * nt, axis=0, keepdims=True)

    xa = xp_ref[0][:, 0:1]
    xb = xp_ref[0][:, 1:2]
    xc = xn_ref[0][:, 0:1]
    lr = lr_ref[0, 0]
    es = es_ref[0, 0]

    def _sample(x, xnext, j):
        n = n_scr[...]
        nx = nx_scr[...]
        xnorm = jnp.sum(x * x)
        d2 = jnp.maximum(norm_scr[...] - 2.0 * nx + xnorm, 0.0)
        d2_scr[...] = d2

        m = jnp.min(d2)
        lane = lax.broadcasted_iota(jnp.int32, (1, _NUM_NEURONS), 1)
        bmu = jnp.min(jnp.where(d2 == m, lane, _NUM_NEURONS))
        row = bmu >> 6
        col = bmu & (_OUT_W - 1)

        es_m = es * m
        r_max = jnp.sqrt(_NEGLIGIBLE_Z * es_m).astype(jnp.int32)
        fits = jnp.logical_and(m > jnp.float32(0.0),
                               2 * r_max + 2 <= _WIN_ROWS)

        def _win_start():
            r0 = jnp.clip(row - r_max, 0, _OUT_H - _WIN_ROWS) & ~1
            return pl.multiple_of(r0 * _OUT_W, 2 * _OUT_W)

        def _neigh_update(sub_lane, nw, d2w):
            kr = sub_lane >> 6
            kc = sub_lane & (_OUT_W - 1)
            gd = (jnp.abs(row - kr) + jnp.abs(col - kc)).astype(jnp.float32)
            neigh = jnp.where(m == jnp.float32(0.0),
                              jnp.zeros_like(d2w),
                              jnp.exp(-(gd * gd) / es_m))
            c = lr * (jnp.sqrt(d2w) * neigh)
            return nw + c * (x - nw)

        p_next = jnp.sum(xnext * n, axis=0, keepdims=True)
        nx_scr[...] = p_next

        @pl.when(fits)
        def _windowed_update():
            s = _win_start()
            lanes = lax.broadcasted_iota(jnp.int32, (1, _WIN), 1) + s
            nw = n_scr[:, pl.ds(s, _WIN)]
            d2w = d2_scr[:, pl.ds(s, _WIN)]
            nw_new = _neigh_update(lanes, nw, d2w)
            n_scr[:, pl.ds(s, _WIN)] = nw_new
            norm_scr[:, pl.ds(s, _WIN)] = jnp.sum(nw_new * nw_new, axis=0,
                                                  keepdims=True)
            nx_scr[:, pl.ds(s, _WIN)] = jnp.sum(xnext * nw_new, axis=0,
                                                keepdims=True)

        @pl.when(jnp.logical_not(fits))
        def _dense_update():
            n_new = _neigh_update(lane, n, d2)
            n_scr[...] = n_new
            norm_scr[...] = jnp.sum(n_new * n_new, axis=0, keepdims=True)
            nx_scr[...] = jnp.sum(xnext * n_new, axis=0, keepdims=True)

        locs_ref[0, j, 0] = row
        locs_ref[0, j, 1] = col
        bmu_ref[0, j, 0] = bmu

    _sample(xa, xb, 0)
    _sample(xb, xc, 1)

    @pl.when(i == pl.num_programs(0) - 1)
    def _finish():
        nout_ref[...] = n_scr[...].T


def _dsom_scan(xp, nT, lr, es):
    nb, d, _ = xp.shape
    k = nT.shape[1]
    return pl.pallas_call(
        _dsom_step_body,
        grid=(nb,),
        in_specs=[
            pl.BlockSpec((1, d, 2), lambda t: (t, 0, 0)),
            pl.BlockSpec((1, d, 2), lambda t: (jnp.minimum(t + 1, nb - 1), 0, 0)),
            pl.BlockSpec((d, k), lambda t: (0, 0)),
            pl.BlockSpec(memory_space=pltpu.SMEM),
            pl.BlockSpec(memory_space=pltpu.SMEM),
        ],
        out_specs=[
            pl.BlockSpec((1, 2, 2), lambda t: (t, 0, 0), memory_space=pltpu.SMEM),
            pl.BlockSpec((1, 2, 1), lambda t: (t, 0, 0), memory_space=pltpu.SMEM),
            pl.BlockSpec((k, d), lambda t: (0, 0)),
        ],
        out_shape=[
            jax.ShapeDtypeStruct((nb, 2, 2), jnp.int32),
            jax.ShapeDtypeStruct((nb, 2, 1), jnp.int32),
            jax.ShapeDtypeStruct((k, d), jnp.float32),
        ],
        scratch_shapes=[pltpu.VMEM((d, k), jnp.float32),
                        pltpu.VMEM((1, k), jnp.float32),
                        pltpu.VMEM((1, k), jnp.float32),
                        pltpu.VMEM((1, k), jnp.float32)],
    )(xp, xp, nT, lr, es)


def _sc_gather(table, idx):
    info = plsc.get_sparse_core_info()
    nw = info.num_cores * info.num_subcores
    b = idx.shape[0]
    d = table.shape[1]
    b_per_w = b // nw
    mesh = plsc.VectorSubcoreMesh(core_axis_name="c", subcore_axis_name="s")

    @functools.partial(
        pl.kernel, mesh=mesh,
        out_type=jax.ShapeDtypeStruct((b, d), jnp.float32),
        scratch_types=[
            pltpu.VMEM((b_per_w,), jnp.int32),
            pltpu.VMEM((b_per_w, d), jnp.float32),
            pltpu.SemaphoreType.DMA,
        ],
    )
    def gather_kernel(table_hbm, idx_hbm, out_hbm, idx_v, rows_v, sem):
        wid = lax.axis_index("s") * info.num_cores + lax.axis_index("c")
        base = wid * b_per_w
        pltpu.sync_copy(idx_hbm.at[pl.ds(base, b_per_w)], idx_v)
        pltpu.async_copy(table_hbm.at[idx_v], rows_v, sem).wait()
        pltpu.sync_copy(rows_v, out_hbm.at[pl.ds(base, b_per_w)])

    return gather_kernel(table, idx)


@jax.jit
def kernel(input, neurons, learning_rate, elasticity_squared):
    b, d = input.shape
    xp = input.reshape(b // 2, 2, d).transpose(0, 2, 1)
    nT = neurons.T
    lr = jnp.asarray(learning_rate, jnp.float32).reshape(1, 1)
    es = jnp.asarray(elasticity_squared, jnp.float32).reshape(1, 1)

    locs, bmus, n_final = _dsom_scan(xp, nT, lr, es)
    values = _sc_gather(n_final, bmus.reshape(b))
    return locs.reshape(b, 2), values


# speedup vs baseline: 1.3925x; 1.1249x over previous
import functools

import jax
import jax.numpy as jnp
from jax import lax
from jax.experimental import pallas as pl
from jax.experimental.pallas import tpu as pltpu
from jax.experimental.pallas import tpu_sc as plsc

_OUT_H = 64
_OUT_W = 64
_NUM_NEURONS = _OUT_H * _OUT_W

_WIN_ROWS = 4
_WIN = _WIN_ROWS * _OUT_W
_NEGLIGIBLE_Z = 40.0


def _dsom_step_body(xp_ref, xn_ref, nT_ref, lr_ref, es_ref,
                    locs_ref, bmu_ref, nout_ref,
                    n_scr, norm_scr, nx_scr, d2_scr):
    i = pl.program_id(0)

    @pl.when(i == 0)
    def _init():
        nt = nT_ref[...]
        n_scr[...] = nt
        norm_scr[...] = jnp.sum(nt * nt, axis=0, keepdims=True)
        nx_scr[...] = jnp.sum(xp_ref[0][:, 0:1] * nt, axis=0, keepdims=True)

    xa = xp_ref[0][:, 0:1]
    xb = xp_ref[0][:, 1:2]
    xc = xn_ref[0][:, 0:1]
    lr = lr_ref[0, 0]
    es = es_ref[0, 0]

    def _sample(x, xnext, j):
        n = n_scr[...]
        nx = nx_scr[...]
        xnorm = jnp.sum(x * x)
        dm = norm_scr[...] - 2.0 * nx
        d2_scr[...] = dm

        mm = jnp.min(dm)
        lane = lax.broadcasted_iota(jnp.int32, (1, _NUM_NEURONS), 1)
        bmu = jnp.min(jnp.where(dm == mm, lane, _NUM_NEURONS))
        row = bmu >> 6
        col = bmu & (_OUT_W - 1)
        m = jnp.maximum(mm + xnorm, 0.0)

        es_m = es * m
        r_max = jnp.sqrt(_NEGLIGIBLE_Z * es_m).astype(jnp.int32)
        fits = jnp.logical_and(m > jnp.float32(0.0),
                               2 * r_max + 2 <= _WIN_ROWS)

        def _win_start():
            r0 = jnp.clip(row - r_max, 0, _OUT_H - _WIN_ROWS) & ~1
            return pl.multiple_of(r0 * _OUT_W, 2 * _OUT_W)

        def _neigh_update(sub_lane, nw, d2w):
            kr = sub_lane >> 6
            kc = sub_lane & (_OUT_W - 1)
            gd = (jnp.abs(row - kr) + jnp.abs(col - kc)).astype(jnp.float32)
            neigh = jnp.where(m == jnp.float32(0.0),
                              jnp.zeros_like(d2w),
                              jnp.exp(-(gd * gd) / es_m))
            c = lr * (jnp.sqrt(d2w) * neigh)
            return nw + c * (x - nw)

        p_next = jnp.sum(xnext * n, axis=0, keepdims=True)
        nx_scr[...] = p_next

        @pl.when(fits)
        def _windowed_update():
            s = _win_start()
            lanes = lax.broadcasted_iota(jnp.int32, (1, _WIN), 1) + s
            nw = n_scr[:, pl.ds(s, _WIN)]
            d2w = jnp.maximum(d2_scr[:, pl.ds(s, _WIN)] + xnorm, 0.0)
            nw_new = _neigh_update(lanes, nw, d2w)
            n_scr[:, pl.ds(s, _WIN)] = nw_new
            norm_scr[:, pl.ds(s, _WIN)] = jnp.sum(nw_new * nw_new, axis=0,
                                                  keepdims=True)
            nx_scr[:, pl.ds(s, _WIN)] = jnp.sum(xnext * nw_new, axis=0,
                                                keepdims=True)

        @pl.when(jnp.logical_not(fits))
        def _dense_update():
            n_new = _neigh_update(lane, n, jnp.maximum(dm + xnorm, 0.0))
            n_scr[...] = n_new
            norm_scr[...] = jnp.sum(n_new * n_new, axis=0, keepdims=True)
            nx_scr[...] = jnp.sum(xnext * n_new, axis=0, keepdims=True)

        locs_ref[0, j, 0] = row
        locs_ref[0, j, 1] = col
        bmu_ref[0, j, 0] = bmu

    _sample(xa, xb, 0)
    _sample(xb, xc, 1)

    @pl.when(i == pl.num_programs(0) - 1)
    def _finish():
        nout_ref[...] = n_scr[...].T


def _dsom_scan(xp, nT, lr, es):
    nb, d, _ = xp.shape
    k = nT.shape[1]
    return pl.pallas_call(
        _dsom_step_body,
        grid=(nb,),
        in_specs=[
            pl.BlockSpec((1, d, 2), lambda t: (t, 0, 0)),
            pl.BlockSpec((1, d, 2), lambda t: (jnp.minimum(t + 1, nb - 1), 0, 0)),
            pl.BlockSpec((d, k), lambda t: (0, 0)),
            pl.BlockSpec(memory_space=pltpu.SMEM),
            pl.BlockSpec(memory_space=pltpu.SMEM),
        ],
        out_specs=[
            pl.BlockSpec((1, 2, 2), lambda t: (t, 0, 0), memory_space=pltpu.SMEM),
            pl.BlockSpec((1, 2, 1), lambda t: (t, 0, 0), memory_space=pltpu.SMEM),
            pl.BlockSpec((k, d), lambda t: (0, 0)),
        ],
        out_shape=[
            jax.ShapeDtypeStruct((nb, 2, 2), jnp.int32),
            jax.ShapeDtypeStruct((nb, 2, 1), jnp.int32),
            jax.ShapeDtypeStruct((k, d), jnp.float32),
        ],
        scratch_shapes=[pltpu.VMEM((d, k), jnp.float32),
                        pltpu.VMEM((1, k), jnp.float32),
                        pltpu.VMEM((1, k), jnp.float32),
                        pltpu.VMEM((1, k), jnp.float32)],
    )(xp, xp, nT, lr, es)


def _sc_gather(table, idx):
    info = plsc.get_sparse_core_info()
    nw = info.num_cores * info.num_subcores
    b = idx.shape[0]
    d = table.shape[1]
    b_per_w = b // nw
    mesh = plsc.VectorSubcoreMesh(core_axis_name="c", subcore_axis_name="s")

    @functools.partial(
        pl.kernel, mesh=mesh,
        out_type=jax.ShapeDtypeStruct((b, d), jnp.float32),
        scratch_types=[
            pltpu.VMEM((b_per_w,), jnp.int32),
            pltpu.VMEM((b_per_w, d), jnp.float32),
            pltpu.SemaphoreType.DMA,
        ],
    )
    def gather_kernel(table_hbm, idx_hbm, out_hbm, idx_v, rows_v, sem):
        wid = lax.axis_index("s") * info.num_cores + lax.axis_index("c")
        base = wid * b_per_w
        pltpu.sync_copy(idx_hbm.at[pl.ds(base, b_per_w)], idx_v)
        pltpu.async_copy(table_hbm.at[idx_v], rows_v, sem).wait()
        pltpu.sync_copy(rows_v, out_hbm.at[pl.ds(base, b_per_w)])

    return gather_kernel(table, idx)


@jax.jit
def kernel(input, neurons, learning_rate, elasticity_squared):
    b, d = input.shape
    xp = input.reshape(b // 2, 2, d).transpose(0, 2, 1)
    nT = neurons.T
    lr = jnp.asarray(learning_rate, jnp.float32).reshape(1, 1)
    es = jnp.asarray(elasticity_squared, jnp.float32).reshape(1, 1)

    locs, bmus, n_final = _dsom_scan(xp, nT, lr, es)
    values = _sc_gather(n_final, bmus.reshape(b))
    return locs.reshape(b, 2), values
